# baseline (speedup 1.0000x reference)
import functools

import jax
import jax.numpy as jnp
from jax import lax
from jax.experimental import pallas as pl
from jax.experimental.pallas import tpu as pltpu
from jax.experimental.pallas import tpu_sc as plsc

D_MODEL = 1024
D_FF = 2048
N_EXPERTS = 8
BM = 128
R = 2 * 2048
R_PAD = R + N_EXPERTS * BM
G = R_PAD // BM


def _gating_kernel(logits_ref, pos0_ref, pos1_ref, g1_ref, g2_ref, te_ref):
    logits = logits_ref[...]
    n = logits.shape[0]
    e = jax.lax.broadcasted_iota(jnp.int32, logits.shape, 1)
    l1 = jnp.max(logits, axis=1, keepdims=True)
    i1 = jnp.min(jnp.where(logits == l1, e, N_EXPERTS), axis=1, keepdims=True)
    masked = jnp.where(e == i1, -jnp.inf, logits)
    l2 = jnp.max(masked, axis=1, keepdims=True)
    i2 = jnp.min(jnp.where(masked == l2, e, N_EXPERTS), axis=1, keepdims=True)
    t = jnp.exp(l2 - l1)
    g2 = t / (1.0 + t)
    g1_ref[...] = 1.0 - g2
    g2_ref[...] = g2

    oh1 = (e == i1).astype(jnp.bfloat16)
    oh2 = (e == i2).astype(jnp.bfloat16)
    both = oh1 + oh2
    r = jax.lax.broadcasted_iota(jnp.int32, (n, n), 0)
    c = jax.lax.broadcasted_iota(jnp.int32, (n, n), 1)
    tril = (r > c).astype(jnp.bfloat16)
    prefix = jnp.dot(tril, both, preferred_element_type=jnp.float32)
    counts = jnp.sum(both.astype(jnp.float32), axis=0, keepdims=True)
    padded = jnp.floor((counts + (BM - 1)) * (1.0 / BM)).astype(jnp.float32)
    padded = padded * BM
    e8r = jax.lax.broadcasted_iota(jnp.int32, (N_EXPERTS, N_EXPERTS), 0)
    e8c = jax.lax.broadcasted_iota(jnp.int32, (N_EXPERTS, N_EXPERTS), 1)
    incl = (e8r <= e8c).astype(jnp.bfloat16)
    pad_cum = jnp.dot(padded.astype(jnp.bfloat16), incl,
                      preferred_element_type=jnp.float32)
    poff = pad_cum - padded
    slot = poff + prefix
    pos0_ref[...] = jnp.sum(oh1.astype(jnp.float32) * slot, axis=1,
                            keepdims=True).astype(jnp.int32)
    pos1_ref[...] = jnp.sum(oh2.astype(jnp.float32) * slot, axis=1,
                            keepdims=True).astype(jnp.int32)

    tv = (jax.lax.broadcasted_iota(jnp.int32, (64, N_EXPERTS), 0)
          * BM).astype(jnp.float32)
    te = jnp.sum((tv >= pad_cum).astype(jnp.int32), axis=1, keepdims=True)
    te_ref[...] = jnp.minimum(te, N_EXPERTS - 1)


def _ffn_kernel(te_ref, xs_ref, b1_ref, b2_ref, w1_hbm, w2_hbm,
                o_ref, w1f_ref, w2f_ref, w1b_ref, w2b_ref, sem1, sem2):
    t = pl.program_id(0)
    cur = te_ref[t]
    prev = jnp.where(t == 0, -1, te_ref[jnp.maximum(t - 1, 0)])

    def _issue(e):
        s = jax.lax.rem(e, 2)
        pltpu.make_async_copy(w1_hbm.at[e], w1f_ref.at[s], sem1.at[e]).start()
        pltpu.make_async_copy(w2_hbm.at[e], w2f_ref.at[s], sem2.at[e]).start()

    def _wait(e):
        s = jax.lax.rem(e, 2)
        pltpu.make_async_copy(w1_hbm.at[e], w1f_ref.at[s], sem1.at[e]).wait()
        pltpu.make_async_copy(w2_hbm.at[e], w2f_ref.at[s], sem2.at[e]).wait()

    @pl.when(cur != prev)
    def _():
        @pl.when(t == 0)
        def _():
            _issue(cur)

        @pl.when(jnp.logical_and(t > 0, cur != prev + 1))
        def _():
            _wait(prev + 1)
            _issue(cur)

        _wait(cur)

        @pl.when(cur < N_EXPERTS - 1)
        def _():
            _issue(cur + 1)

        s = jax.lax.rem(cur, 2)
        w1b_ref[...] = w1f_ref[s].astype(jnp.bfloat16)
        w2b_ref[...] = w2f_ref[s].astype(jnp.bfloat16)

    xs = xs_ref[...].astype(jnp.bfloat16)
    h = jnp.dot(xs, w1b_ref[...], preferred_element_type=jnp.float32)
    h = jnp.maximum(h + b1_ref[0], 0.0).astype(jnp.bfloat16)
    o = jnp.dot(h, w2b_ref[...], preferred_element_type=jnp.float32)
    o_ref[...] = (o + b2_ref[0]).astype(jnp.bfloat16)


def _grouped_ffn(tile_expert, Xs, W1, b1, W2, b2):
    return pl.pallas_call(
        _ffn_kernel,
        grid_spec=pltpu.PrefetchScalarGridSpec(
            num_scalar_prefetch=1,
            grid=(G,),
            in_specs=[
                pl.BlockSpec((BM, D_MODEL), lambda t, te: (t, 0)),
                pl.BlockSpec((1, 1, D_FF), lambda t, te: (te[t], 0, 0)),
                pl.BlockSpec((1, 1, D_MODEL), lambda t, te: (te[t], 0, 0)),
                pl.BlockSpec(memory_space=pl.ANY),
                pl.BlockSpec(memory_space=pl.ANY),
            ],
            out_specs=pl.BlockSpec((BM, D_MODEL), lambda t, te: (t, 0)),
            scratch_shapes=[
                pltpu.VMEM((2, D_MODEL, D_FF), jnp.float32),
                pltpu.VMEM((2, D_FF, D_MODEL), jnp.float32),
                pltpu.VMEM((D_MODEL, D_FF), jnp.bfloat16),
                pltpu.VMEM((D_FF, D_MODEL), jnp.bfloat16),
                pltpu.SemaphoreType.DMA((N_EXPERTS,)),
                pltpu.SemaphoreType.DMA((N_EXPERTS,)),
            ],
        ),
        out_shape=jax.ShapeDtypeStruct((R_PAD, D_MODEL), jnp.bfloat16),
    )(tile_expert, Xs,
      b1.reshape(N_EXPERTS, 1, D_FF), b2.reshape(N_EXPERTS, 1, D_MODEL),
      W1, W2)


def _dispatch_body(x_hbm, pos0_hbm, pos1_hbm, xs_out,
                   p0_v, p1_v, rows_v, sem):
    core = lax.axis_index("c")
    sub = lax.axis_index("s")
    ch = p0_v.shape[0]
    wid = sub * 2 + core
    base = wid * ch
    pltpu.sync_copy(pos0_hbm.at[pl.ds(base, ch)], p0_v)
    pltpu.sync_copy(pos1_hbm.at[pl.ds(base, ch)], p1_v)
    pltpu.sync_copy(x_hbm.at[pl.ds(base, ch)], rows_v)
    c1 = pltpu.async_copy(rows_v, xs_out.at[p0_v], sem)
    c2 = pltpu.async_copy(rows_v, xs_out.at[p1_v], sem)
    c1.wait()
    c2.wait()


def _sc_dispatch(x, pos0f, pos1f, n):
    ch = n // 32
    disp = pl.kernel(
        _dispatch_body,
        out_type=jax.ShapeDtypeStruct((R_PAD, D_MODEL), jnp.float32),
        mesh=plsc.VectorSubcoreMesh(core_axis_name="c", subcore_axis_name="s"),
        scratch_types=[
            pltpu.VMEM((ch,), jnp.int32),
            pltpu.VMEM((ch,), jnp.int32),
            pltpu.VMEM((ch, D_MODEL), jnp.float32),
            pltpu.SemaphoreType.DMA,
        ],
    )
    return disp(x, pos0f, pos1f)


def kernel(x, W1, b1, W2, b2, Wg, bg):
    n = x.shape[0]
    logits = x @ Wg + bg
    pos0, pos1, g1, g2, te = pl.pallas_call(
        _gating_kernel,
        out_shape=[
            jax.ShapeDtypeStruct((n, 1), jnp.int32),
            jax.ShapeDtypeStruct((n, 1), jnp.int32),
            jax.ShapeDtypeStruct((n, 1), jnp.float32),
            jax.ShapeDtypeStruct((n, 1), jnp.float32),
            jax.ShapeDtypeStruct((64, 1), jnp.int32),
        ],
        in_specs=[pl.BlockSpec((n, N_EXPERTS), lambda: (0, 0))],
        out_specs=[pl.BlockSpec((n, 1), lambda: (0, 0))] * 4
        + [pl.BlockSpec((64, 1), lambda: (0, 0))],
    )(logits)
    tile_expert = te.reshape(-1)
    p0f = pos0.reshape(-1)
    p1f = pos1.reshape(-1)

    Xs = _sc_dispatch(x, p0f, p1f, n)

    O = _grouped_ffn(tile_expert, Xs, W1, b1, W2, b2)

    out = g1 * O[p0f].astype(jnp.float32) + g2 * O[p1f].astype(jnp.float32)
    return out


# speedup vs baseline: 1.0235x; 1.0235x over previous
import functools

import jax
import jax.numpy as jnp
from jax import lax
from jax.experimental import pallas as pl
from jax.experimental.pallas import tpu as pltpu
from jax.experimental.pallas import tpu_sc as plsc

D_MODEL = 1024
D_FF = 2048
N_EXPERTS = 8
BM = 128
R = 2 * 2048
R_PAD = R + N_EXPERTS * BM
G = R_PAD // BM


def _gating_kernel(logits_ref, pos0_ref, pos1_ref, g1_ref, g2_ref, te_ref):
    logits = logits_ref[...]
    n = logits.shape[0]
    e = jax.lax.broadcasted_iota(jnp.int32, logits.shape, 1)
    l1 = jnp.max(logits, axis=1, keepdims=True)
    i1 = jnp.min(jnp.where(logits == l1, e, N_EXPERTS), axis=1, keepdims=True)
    masked = jnp.where(e == i1, -jnp.inf, logits)
    l2 = jnp.max(masked, axis=1, keepdims=True)
    i2 = jnp.min(jnp.where(masked == l2, e, N_EXPERTS), axis=1, keepdims=True)
    t = jnp.exp(l2 - l1)
    g2 = t / (1.0 + t)
    g1_ref[...] = 1.0 - g2
    g2_ref[...] = g2

    oh1 = (e == i1).astype(jnp.bfloat16)
    oh2 = (e == i2).astype(jnp.bfloat16)
    both = oh1 + oh2
    r = jax.lax.broadcasted_iota(jnp.int32, (n, n), 0)
    c = jax.lax.broadcasted_iota(jnp.int32, (n, n), 1)
    tril = (r > c).astype(jnp.bfloat16)
    prefix = jnp.dot(tril, both, preferred_element_type=jnp.float32)
    counts = jnp.sum(both.astype(jnp.float32), axis=0, keepdims=True)
    padded = jnp.floor((counts + (BM - 1)) * (1.0 / BM)).astype(jnp.float32)
    padded = padded * BM
    e8r = jax.lax.broadcasted_iota(jnp.int32, (N_EXPERTS, N_EXPERTS), 0)
    e8c = jax.lax.broadcasted_iota(jnp.int32, (N_EXPERTS, N_EXPERTS), 1)
    incl = (e8r <= e8c).astype(jnp.bfloat16)
    pad_cum = jnp.dot(padded.astype(jnp.bfloat16), incl,
                      preferred_element_type=jnp.float32)
    poff = pad_cum - padded
    slot = poff + prefix
    pos0_ref[...] = jnp.sum(oh1.astype(jnp.float32) * slot, axis=1,
                            keepdims=True).astype(jnp.int32)
    pos1_ref[...] = jnp.sum(oh2.astype(jnp.float32) * slot, axis=1,
                            keepdims=True).astype(jnp.int32)

    tv = (jax.lax.broadcasted_iota(jnp.int32, (64, N_EXPERTS), 0)
          * BM).astype(jnp.float32)
    te = jnp.sum((tv >= pad_cum).astype(jnp.int32), axis=1, keepdims=True)
    te_ref[...] = jnp.minimum(te, N_EXPERTS - 1)


def _ffn_kernel(te_ref, xs_ref, b1_ref, b2_ref, w1_hbm, w2_hbm,
                o_ref, w1f_ref, w2f_ref, w1b_ref, w2b_ref, sem1, sem2):
    t = pl.program_id(0)
    cur = te_ref[t]
    prev = jnp.where(t == 0, -1, te_ref[jnp.maximum(t - 1, 0)])

    def _issue(e):
        s = jax.lax.rem(e, 2)
        pltpu.make_async_copy(w1_hbm.at[e], w1f_ref.at[s], sem1.at[e]).start()
        pltpu.make_async_copy(w2_hbm.at[e], w2f_ref.at[s], sem2.at[e]).start()

    def _wait(e):
        s = jax.lax.rem(e, 2)
        pltpu.make_async_copy(w1_hbm.at[e], w1f_ref.at[s], sem1.at[e]).wait()
        pltpu.make_async_copy(w2_hbm.at[e], w2f_ref.at[s], sem2.at[e]).wait()

    @pl.when(cur != prev)
    def _():
        @pl.when(t == 0)
        def _():
            _issue(cur)

        @pl.when(jnp.logical_and(t > 0, cur != prev + 1))
        def _():
            _wait(prev + 1)
            _issue(cur)

        _wait(cur)

        @pl.when(cur < N_EXPERTS - 1)
        def _():
            _issue(cur + 1)

        s = jax.lax.rem(cur, 2)
        w1b_ref[...] = w1f_ref[s].astype(jnp.bfloat16)
        w2b_ref[...] = w2f_ref[s].astype(jnp.bfloat16)

    xs = xs_ref[...].astype(jnp.bfloat16)
    hf = D_FF // 2
    h1 = jnp.dot(xs, w1b_ref[:, :hf], preferred_element_type=jnp.float32)
    h1 = jnp.maximum(h1 + b1_ref[0, :, :hf], 0.0).astype(jnp.bfloat16)
    h2 = jnp.dot(xs, w1b_ref[:, hf:], preferred_element_type=jnp.float32)
    o1 = jnp.dot(h1, w2b_ref[:hf, :], preferred_element_type=jnp.float32)
    h2 = jnp.maximum(h2 + b1_ref[0, :, hf:], 0.0).astype(jnp.bfloat16)
    o2 = jnp.dot(h2, w2b_ref[hf:, :], preferred_element_type=jnp.float32)
    o_ref[...] = o1 + o2 + b2_ref[0]


def _grouped_ffn(tile_expert, Xs, W1, b1, W2, b2):
    return pl.pallas_call(
        _ffn_kernel,
        grid_spec=pltpu.PrefetchScalarGridSpec(
            num_scalar_prefetch=1,
            grid=(G,),
            in_specs=[
                pl.BlockSpec((BM, D_MODEL), lambda t, te: (t, 0)),
                pl.BlockSpec((1, 1, D_FF), lambda t, te: (te[t], 0, 0)),
                pl.BlockSpec((1, 1, D_MODEL), lambda t, te: (te[t], 0, 0)),
                pl.BlockSpec(memory_space=pl.ANY),
                pl.BlockSpec(memory_space=pl.ANY),
            ],
            out_specs=pl.BlockSpec((BM, D_MODEL), lambda t, te: (t, 0)),
            scratch_shapes=[
                pltpu.VMEM((2, D_MODEL, D_FF), jnp.float32),
                pltpu.VMEM((2, D_FF, D_MODEL), jnp.float32),
                pltpu.VMEM((D_MODEL, D_FF), jnp.bfloat16),
                pltpu.VMEM((D_FF, D_MODEL), jnp.bfloat16),
                pltpu.SemaphoreType.DMA((N_EXPERTS,)),
                pltpu.SemaphoreType.DMA((N_EXPERTS,)),
            ],
        ),
        out_shape=jax.ShapeDtypeStruct((R_PAD, D_MODEL), jnp.float32),
    )(tile_expert, Xs,
      b1.reshape(N_EXPERTS, 1, D_FF), b2.reshape(N_EXPERTS, 1, D_MODEL),
      W1, W2)


def _dispatch_body(x_hbm, pos0_hbm, pos1_hbm, xs_out,
                   p0_v, p1_v, rows_v, sem):
    core = lax.axis_index("c")
    sub = lax.axis_index("s")
    ch = p0_v.shape[0]
    wid = sub * 2 + core
    base = wid * ch
    pltpu.sync_copy(pos0_hbm.at[pl.ds(base, ch)], p0_v)
    pltpu.sync_copy(pos1_hbm.at[pl.ds(base, ch)], p1_v)
    pltpu.sync_copy(x_hbm.at[pl.ds(base, ch)], rows_v)
    c1 = pltpu.async_copy(rows_v, xs_out.at[p0_v], sem)
    c2 = pltpu.async_copy(rows_v, xs_out.at[p1_v], sem)
    c1.wait()
    c2.wait()


def _sc_dispatch(x, pos0f, pos1f, n):
    ch = n // 32
    disp = pl.kernel(
        _dispatch_body,
        out_type=jax.ShapeDtypeStruct((R_PAD, D_MODEL), jnp.float32),
        mesh=plsc.VectorSubcoreMesh(core_axis_name="c", subcore_axis_name="s"),
        scratch_types=[
            pltpu.VMEM((ch,), jnp.int32),
            pltpu.VMEM((ch,), jnp.int32),
            pltpu.VMEM((ch, D_MODEL), jnp.float32),
            pltpu.SemaphoreType.DMA,
        ],
    )
    return disp(x, pos0f, pos1f)


def kernel(x, W1, b1, W2, b2, Wg, bg):
    n = x.shape[0]
    logits = x @ Wg + bg
    pos0, pos1, g1, g2, te = pl.pallas_call(
        _gating_kernel,
        out_shape=[
            jax.ShapeDtypeStruct((n, 1), jnp.int32),
            jax.ShapeDtypeStruct((n, 1), jnp.int32),
            jax.ShapeDtypeStruct((n, 1), jnp.float32),
            jax.ShapeDtypeStruct((n, 1), jnp.float32),
            jax.ShapeDtypeStruct((64, 1), jnp.int32),
        ],
        in_specs=[pl.BlockSpec((n, N_EXPERTS), lambda: (0, 0))],
        out_specs=[pl.BlockSpec((n, 1), lambda: (0, 0))] * 4
        + [pl.BlockSpec((64, 1), lambda: (0, 0))],
    )(logits)
    tile_expert = te.reshape(-1)
    p0f = pos0.reshape(-1)
    p1f = pos1.reshape(-1)

    Xs = _sc_dispatch(x, p0f, p1f, n)

    O = _grouped_ffn(tile_expert, Xs, W1, b1, W2, b2)

    out = g1 * O[p0f] + g2 * O[p1f]
    return out


# speedup vs baseline: 1.0309x; 1.0072x over previous
import jax
import jax.numpy as jnp
from jax import lax
from jax.experimental import pallas as pl
from jax.experimental.pallas import tpu as pltpu
from jax.experimental.pallas import tpu_sc as plsc

D_MODEL = 1024
D_FF = 2048
N_EXPERTS = 8
BM = 128
R = 2 * 2048
R_PAD = R + N_EXPERTS * BM
G = R_PAD // BM


def _gating_kernel(logits_ref, pos0_ref, pos1_ref, g1_ref, g2_ref, te_ref):
    logits = logits_ref[...]
    n = logits.shape[0]
    e = jax.lax.broadcasted_iota(jnp.int32, logits.shape, 1)
    l1 = jnp.max(logits, axis=1, keepdims=True)
    i1 = jnp.min(jnp.where(logits == l1, e, N_EXPERTS), axis=1, keepdims=True)
    masked = jnp.where(e == i1, -jnp.inf, logits)
    l2 = jnp.max(masked, axis=1, keepdims=True)
    i2 = jnp.min(jnp.where(masked == l2, e, N_EXPERTS), axis=1, keepdims=True)
    t = jnp.exp(l2 - l1)
    g2 = t / (1.0 + t)
    g1_ref[...] = 1.0 - g2
    g2_ref[...] = g2

    oh1 = (e == i1).astype(jnp.bfloat16)
    oh2 = (e == i2).astype(jnp.bfloat16)
    both = oh1 + oh2
    r = jax.lax.broadcasted_iota(jnp.int32, (n, n), 0)
    c = jax.lax.broadcasted_iota(jnp.int32, (n, n), 1)
    tril = (r > c).astype(jnp.bfloat16)
    prefix = jnp.dot(tril, both, preferred_element_type=jnp.float32)
    counts = jnp.sum(both.astype(jnp.float32), axis=0, keepdims=True)
    padded = jnp.floor((counts + (BM - 1)) * (1.0 / BM)).astype(jnp.float32)
    padded = padded * BM
    e8r = jax.lax.broadcasted_iota(jnp.int32, (N_EXPERTS, N_EXPERTS), 0)
    e8c = jax.lax.broadcasted_iota(jnp.int32, (N_EXPERTS, N_EXPERTS), 1)
    incl = (e8r <= e8c).astype(jnp.bfloat16)
    pad_cum = jnp.dot(padded.astype(jnp.bfloat16), incl,
                      preferred_element_type=jnp.float32)
    poff = pad_cum - padded
    slot = poff + prefix
    pos0_ref[...] = jnp.sum(oh1.astype(jnp.float32) * slot, axis=1,
                            keepdims=True).astype(jnp.int32)
    pos1_ref[...] = jnp.sum(oh2.astype(jnp.float32) * slot, axis=1,
                            keepdims=True).astype(jnp.int32)

    tv = (jax.lax.broadcasted_iota(jnp.int32, (64, N_EXPERTS), 0)
          * BM).astype(jnp.float32)
    te = jnp.sum((tv >= pad_cum).astype(jnp.int32), axis=1, keepdims=True)
    te_ref[...] = jnp.minimum(te, N_EXPERTS - 1)


def _ffn_kernel(te_ref, xs_ref, b1_ref, b2_ref, w1_hbm, w2_hbm,
                o_ref, w1f_ref, w2f_ref, w1b_ref, w2b_ref, sem1, sem2):
    t = pl.program_id(0)
    cur = te_ref[t]
    prev = jnp.where(t == 0, -1, te_ref[jnp.maximum(t - 1, 0)])

    def _issue(e):
        s = jax.lax.rem(e, 2)
        pltpu.make_async_copy(w1_hbm.at[e], w1f_ref.at[s], sem1.at[e]).start()
        pltpu.make_async_copy(w2_hbm.at[e], w2f_ref.at[s], sem2.at[e]).start()

    def _wait(e):
        s = jax.lax.rem(e, 2)
        pltpu.make_async_copy(w1_hbm.at[e], w1f_ref.at[s], sem1.at[e]).wait()
        pltpu.make_async_copy(w2_hbm.at[e], w2f_ref.at[s], sem2.at[e]).wait()

    @pl.when(cur != prev)
    def _():
        @pl.when(t == 0)
        def _():
            _issue(cur)

        @pl.when(jnp.logical_and(t > 0, cur != prev + 1))
        def _():
            _wait(prev + 1)
            _issue(cur)

        _wait(cur)

        @pl.when(cur < N_EXPERTS - 1)
        def _():
            _issue(cur + 1)

        s = jax.lax.rem(cur, 2)
        w1b_ref[...] = w1f_ref[s].astype(jnp.bfloat16)
        w2b_ref[...] = w2f_ref[s].astype(jnp.bfloat16)

    xs = xs_ref[...].astype(jnp.bfloat16)
    h = jnp.dot(xs, w1b_ref[...], preferred_element_type=jnp.float32)
    h = jnp.maximum(h + b1_ref[0], 0.0).astype(jnp.bfloat16)
    o = jnp.dot(h, w2b_ref[...], preferred_element_type=jnp.float32)
    o_ref[...] = o + b2_ref[0]


def _grouped_ffn(tile_expert, Xs, W1, b1, W2, b2):
    return pl.pallas_call(
        _ffn_kernel,
        grid_spec=pltpu.PrefetchScalarGridSpec(
            num_scalar_prefetch=1,
            grid=(G,),
            in_specs=[
                pl.BlockSpec((BM, D_MODEL), lambda t, te: (t, 0)),
                pl.BlockSpec((1, 1, D_FF), lambda t, te: (te[t], 0, 0)),
                pl.BlockSpec((1, 1, D_MODEL), lambda t, te: (te[t], 0, 0)),
                pl.BlockSpec(memory_space=pl.ANY),
                pl.BlockSpec(memory_space=pl.ANY),
            ],
            out_specs=pl.BlockSpec((BM, D_MODEL), lambda t, te: (t, 0)),
            scratch_shapes=[
                pltpu.VMEM((2, D_MODEL, D_FF), jnp.float32),
                pltpu.VMEM((2, D_FF, D_MODEL), jnp.float32),
                pltpu.VMEM((D_MODEL, D_FF), jnp.bfloat16),
                pltpu.VMEM((D_FF, D_MODEL), jnp.bfloat16),
                pltpu.SemaphoreType.DMA((N_EXPERTS,)),
                pltpu.SemaphoreType.DMA((N_EXPERTS,)),
            ],
        ),
        out_shape=jax.ShapeDtypeStruct((R_PAD, D_MODEL), jnp.float32),
    )(tile_expert, Xs,
      b1.reshape(N_EXPERTS, 1, D_FF), b2.reshape(N_EXPERTS, 1, D_MODEL),
      W1, W2)


def _dispatch_body(x_hbm, pos0_hbm, pos1_hbm, xs_out,
                   p0_v, p1_v, rows_v, sem):
    core = lax.axis_index("c")
    sub = lax.axis_index("s")
    ch = p0_v.shape[0]
    wid = sub * 2 + core
    base = wid * ch
    pltpu.sync_copy(pos0_hbm.at[pl.ds(base, ch)], p0_v)
    pltpu.sync_copy(pos1_hbm.at[pl.ds(base, ch)], p1_v)
    pltpu.sync_copy(x_hbm.at[pl.ds(base, ch)], rows_v)
    c1 = pltpu.async_copy(rows_v, xs_out.at[p0_v], sem)
    c2 = pltpu.async_copy(rows_v, xs_out.at[p1_v], sem)
    c1.wait()
    c2.wait()


def _sc_dispatch(x, pos0f, pos1f, n):
    ch = n // 32
    disp = pl.kernel(
        _dispatch_body,
        out_type=jax.ShapeDtypeStruct((R_PAD, D_MODEL), jnp.float32),
        mesh=plsc.VectorSubcoreMesh(core_axis_name="c", subcore_axis_name="s"),
        scratch_types=[
            pltpu.VMEM((ch,), jnp.int32),
            pltpu.VMEM((ch,), jnp.int32),
            pltpu.VMEM((ch, D_MODEL), jnp.float32),
            pltpu.SemaphoreType.DMA,
        ],
    )
    return disp(x, pos0f, pos1f)


def kernel(x, W1, b1, W2, b2, Wg, bg):
    n = x.shape[0]
    logits = x @ Wg + bg
    pos0, pos1, g1, g2, te = pl.pallas_call(
        _gating_kernel,
        out_shape=[
            jax.ShapeDtypeStruct((n, 1), jnp.int32),
            jax.ShapeDtypeStruct((n, 1), jnp.int32),
            jax.ShapeDtypeStruct((n, 1), jnp.float32),
            jax.ShapeDtypeStruct((n, 1), jnp.float32),
            jax.ShapeDtypeStruct((64, 1), jnp.int32),
        ],
        in_specs=[pl.BlockSpec((n, N_EXPERTS), lambda: (0, 0))],
        out_specs=[pl.BlockSpec((n, 1), lambda: (0, 0))] * 4
        + [pl.BlockSpec((64, 1), lambda: (0, 0))],
    )(logits)
    tile_expert = te.reshape(-1)
    p0f = pos0.reshape(-1)
    p1f = pos1.reshape(-1)

    Xs = _sc_dispatch(x, p0f, p1f, n)

    O = _grouped_ffn(tile_expert, Xs, W1, b1, W2, b2)

    out = g1 * O[p0f] + g2 * O[p1f]
    return out
